# baseline (speedup 1.0000x reference)
import functools

import jax
import jax.numpy as jnp
from jax import lax
from jax.experimental import pallas as pl
from jax.experimental.pallas import tpu as pltpu
from jax.experimental.pallas import tpu_sc as plsc

BATCH = 16384
DIM = 32
PAIRS = DIM // 2
N_ROWS = 1000000
NUM_WORKERS = 32
B_PER_W = BATCH // NUM_WORKERS
WPW = PAIRS * B_PER_W
LANES = 16

DT_K = 100
DT_L = DT_K * 128
DT_NC = -(-N_ROWS // DT_L)
OUT_ROWS = DT_NC * 2 * DT_K * 8
FLAT_N = OUT_ROWS * 128


def _tc_detile(tabT):
  grid = (DT_NC,)

  def body(t_ref, o_ref):
    x = t_ref[...]
    lo = jax.lax.bitcast_convert_type(
        x[:PAIRS].astype(jnp.bfloat16), jnp.int16).astype(jnp.int32)
    hi = jax.lax.bitcast_convert_type(
        x[PAIRS:].astype(jnp.bfloat16), jnp.int16).astype(jnp.int32)
    w = (lo & 0xFFFF) | (hi << 16)
    y = w.reshape(2, 8, DT_K, 128).transpose(2, 0, 1, 3)
    o_ref[...] = y.reshape(2 * DT_K * 8, 128)

  return pl.pallas_call(
      body,
      grid=grid,
      in_specs=[pl.BlockSpec((DIM, DT_L), lambda c: (0, c))],
      out_specs=pl.BlockSpec((2 * DT_K * 8, 128), lambda c: (c, 0)),
      out_shape=jax.ShapeDtypeStruct((OUT_ROWS, 128), jnp.int32),
  )(tabT).reshape(FLAT_N)


SC_RUN_T = 16
SC_RUN_L = SC_RUN_T * 128
SC_CHUNK_T = 976
SC_RUNS = SC_CHUNK_T // SC_RUN_T
SC_A2 = 7813 * 1024
SC_FLAT = 2 * SC_A2
SC_TAIL_LANE = 7 * SC_CHUNK_T * 128 + SC_RUNS * SC_RUN_L
SC_TAIL_L = N_ROWS - SC_TAIL_LANE


def _sc_detile_build():
  mesh = plsc.VectorSubcoreMesh(core_axis_name="c", subcore_axis_name="s")
  out_sh = jax.ShapeDtypeStruct((SC_FLAT,), jnp.int32)
  fbuf = pltpu.VMEM((8, SC_RUN_L), jnp.float32)
  wbuf = pltpu.VMEM((SC_RUN_T * 1024,), jnp.int32)

  @functools.partial(
      pl.kernel,
      mesh=mesh,
      compiler_params=pltpu.CompilerParams(needs_layout_passes=False),
      out_type=[out_sh, out_sh],
      scratch_types=[
          fbuf, fbuf, fbuf, fbuf,
          wbuf, wbuf,
          pltpu.SemaphoreType.DMA,
          pltpu.SemaphoreType.DMA,
      ],
  )
  def sc_detile(t0, t1, o0, o1, a0, b0, a1, b1, w0, w1, sem_r, sem_w):
    wid = lax.axis_index("s") * 2 + lax.axis_index("c")
    tab_idx = wid // 16
    rest = wid % 16
    a2 = rest // 8
    chunk = rest % 8
    lane_base = chunk * SC_CHUNK_T * 128
    rowA = a2 * 8
    rowB = a2 * 8 + 16

    def for_tab(tab, src, out):
      @pl.when(tab_idx == tab)
      def _():
        out_base = a2 * SC_A2 + (lane_base // 128) * 1024

        def rd(r, ab, bb):
          lane = lane_base + r * SC_RUN_L
          return (
              pltpu.make_async_copy(
                  src.at[pl.ds(rowA, 8), pl.ds(lane, SC_RUN_L)], ab, sem_r),
              pltpu.make_async_copy(
                  src.at[pl.ds(rowB, 8), pl.ds(lane, SC_RUN_L)], bb, sem_r),
          )

        def wr(r, wb):
          return pltpu.make_async_copy(
              wb, out.at[pl.ds(out_base + r * SC_RUN_T * 1024,
                               SC_RUN_T * 1024)], sem_w)

        def vpass(ab, bb, wb, n_lgroups):
          def vbody(j2, _):
            lane = (j2 >> 3) * 128 + (j2 & 7) * 16
            for s in range(8):
              a = ab[s, pl.ds(lane, LANES)]
              b = bb[s, pl.ds(lane, LANES)]
              w = plsc.bitcast(
                  plsc.pack(a, b, format=plsc.PackFormat.INTERLEAVED,
                            preferred_element_type=jnp.bfloat16),
                  jnp.int32)
              wofs = (j2 >> 3) * 1024 + s * 128 + (j2 & 7) * 16
              wb[pl.ds(wofs, LANES)] = w
            return 0

          lax.fori_loop(0, n_lgroups, vbody, 0, unroll=False)

        for d in rd(0, a0, b0):
          d.start()
        for d in rd(1, a1, b1):
          d.start()

        def body(k, _):
          r0 = 2 * k
          r1 = r0 + 1

          @pl.when(k > 0)
          def _():
            wr(r0 - 2, w0).wait()

          for d in rd(r0, a0, b0):
            d.wait()
          vpass(a0, b0, w0, SC_RUN_T * 8)
          wr(r0, w0).start()

          @pl.when(r0 + 2 < SC_RUNS)
          def _():
            for d in rd(r0 + 2, a0, b0):
              d.start()

          @pl.when(k > 0)
          def _():
            wr(r1 - 2, w1).wait()

          for d in rd(r1, a1, b1):
            d.wait()
          vpass(a1, b1, w1, SC_RUN_T * 8)
          wr(r1, w1).start()

          @pl.when(r1 + 2 < SC_RUNS)
          def _():
            for d in rd(r1 + 2, a1, b1):
              d.start()

          return 0

        lax.fori_loop(0, SC_RUNS // 2, body, 0, unroll=False)
        last = SC_RUNS - 1
        wr(last - 2, w0).wait()
        for d in rd(last, a0, b0):
          d.wait()
        vpass(a0, b0, w0, SC_RUN_T * 8)
        wr(last, w0).start()
        wr(last - 1, w1).wait()
        wr(last, w0).wait()

        @pl.when(chunk == 7)
        def _():
          pltpu.sync_copy(
              src.at[pl.ds(rowA, 8), pl.ds(SC_TAIL_LANE, 512)],
              a1.at[:, pl.ds(0, 512)])
          pltpu.sync_copy(
              src.at[pl.ds(rowB, 8), pl.ds(SC_TAIL_LANE, 512)],
              b1.at[:, pl.ds(0, 512)])
          vpass(a1, b1, w1, 512 // LANES)
          tail_q = SC_TAIL_LANE // 128
          pltpu.sync_copy(
              w1.at[pl.ds(0, 4 * 1024)],
              out.at[pl.ds(a2 * SC_A2 + tail_q * 1024, 4 * 1024)])

    for_tab(0, t0, o0)
    for_tab(1, t1, o1)

  return sc_detile


def _tc_tail(tabT0, tabT1, flat0, flat1):
  q_last = 999936 // 128

  def patch_of(x, g):
    lo = jax.lax.bitcast_convert_type(
        x[:PAIRS].astype(jnp.bfloat16), jnp.int16).astype(jnp.int32)
    hi = jax.lax.bitcast_convert_type(
        x[PAIRS:].astype(jnp.bfloat16), jnp.int16).astype(jnp.int32)
    w = (lo & 0xFFFF) | (hi << 16)
    return jnp.where(g == 0, w[:8].reshape(1024), w[8:].reshape(1024))

  def body(t0_ref, t1_ref, _f0, _f1, o0_ref, o1_ref):
    g = pl.program_id(0)
    o0_ref[...] = patch_of(t0_ref[...], g)
    o1_ref[...] = patch_of(t1_ref[...], g)

  out_sh = jax.ShapeDtypeStruct((SC_FLAT,), jnp.int32)
  out_spec = pl.BlockSpec((1024,), lambda g: (g * (SC_A2 // 1024) + q_last,))
  return pl.pallas_call(
      body,
      grid=(2,),
      in_specs=[
          pl.BlockSpec((DIM, 128), lambda g: (0, q_last)),
          pl.BlockSpec((DIM, 128), lambda g: (0, q_last)),
          pl.BlockSpec(memory_space=pl.ANY),
          pl.BlockSpec(memory_space=pl.ANY),
      ],
      out_specs=[out_spec, out_spec],
      out_shape=[out_sh, out_sh],
      input_output_aliases={2: 0, 3: 1},
  )(tabT0, tabT1, flat0, flat1)


def _sc_gather_build():
  mesh = plsc.VectorSubcoreMesh(core_axis_name="c", subcore_axis_name="s")
  out_sh = jax.ShapeDtypeStruct((NUM_WORKERS, WPW), jnp.int32)
  ebuf = pltpu.VMEM((WPW,), jnp.int32)
  dbuf = pltpu.VMEM((WPW,), jnp.int32)

  @functools.partial(
      pl.kernel,
      mesh=mesh,
      out_type=[out_sh, out_sh, out_sh, out_sh],
      scratch_types=[
          pltpu.VMEM((B_PER_W,), jnp.int32),
          pltpu.VMEM((B_PER_W,), jnp.int32),
          ebuf, ebuf, ebuf, ebuf,
          dbuf, dbuf, dbuf, dbuf,
          pltpu.SemaphoreType.DMA,
      ],
  )
  def sc_gather(user_hbm, item_hbm, g0_hbm, g1_hbm, f2_hbm, f3_hbm,
                o0, o1, o2, o3,
                idx_u, idx_i, eu_sc, ei_sc, eu_tc, ei_tc,
                d0, d1, d2, d3, sem):
    wid = lax.axis_index("s") * 2 + lax.axis_index("c")
    base = wid * B_PER_W
    pltpu.sync_copy(user_hbm.at[pl.ds(base, B_PER_W)], idx_u)
    pltpu.sync_copy(item_hbm.at[pl.ds(base, B_PER_W)], idx_i)

    for j in range(B_PER_W // LANES):
      src = pl.ds(j * LANES, LANES)
      iu = idx_u[src]
      ii = idx_i[src]
      bu_sc = ((iu >> 7) << 10) + (iu & 127)
      bi_sc = ((ii >> 7) << 10) + (ii & 127)
      bu_tc = bu_sc + ((iu >> 7) << 10)
      bi_tc = bi_sc + ((ii >> 7) << 10)
      for p in range(PAIRS):
        off_sc = (p // 8) * SC_A2 + (p % 8) * 128
        off_tc = (p // 8) * 1024 + (p % 8) * 128
        dst = pl.ds(p * B_PER_W + j * LANES, LANES)
        eu_sc[dst] = bu_sc + off_sc
        ei_sc[dst] = bi_sc + off_sc
        eu_tc[dst] = bu_tc + off_tc
        ei_tc[dst] = bi_tc + off_tc

    copies = [
        pltpu.async_copy(g0_hbm.at[eu_sc], d0, sem),
        pltpu.async_copy(g1_hbm.at[ei_sc], d1, sem),
        pltpu.async_copy(f2_hbm.at[eu_tc], d2, sem),
        pltpu.async_copy(f3_hbm.at[ei_tc], d3, sem),
    ]
    for c in copies:
      c.wait()
    for buf, out in ((d0, o0), (d1, o1), (d2, o2), (d3, o3)):
      pltpu.sync_copy(buf, out.at[wid])

  return sc_gather


_BUILD_CACHE = {}


def _get(name, builder):
  if name not in _BUILD_CACHE:
    _BUILD_CACHE[name] = builder()
  return _BUILD_CACHE[name]


def _unpack(w):
  lo = jax.lax.bitcast_convert_type(w << 16, jnp.float32)
  hi = jax.lax.bitcast_convert_type(w & jnp.int32(-65536), jnp.float32)
  return jnp.concatenate([lo, hi], axis=0)


def _tc_body(mfu, mfi, mlu, mli, w1, b1c, w2, b2c, wpa, wpb, bpr, out):
  f32 = jnp.float32
  u = _unpack(mlu[0])
  i = _unpack(mli[0])
  w1m = w1[...]
  dn = (((1,), (0,)), ((), ()))
  x = (lax.dot_general(w1m[:, :DIM], u, dn, preferred_element_type=f32)
       + lax.dot_general(w1m[:, DIM:], i, dn, preferred_element_type=f32)
       + b1c[...])
  h = jnp.maximum(x, 0.0)
  h2 = jnp.maximum(
      lax.dot_general(w2[...], h, dn, preferred_element_type=f32) + b2c[...],
      0.0)
  mfp = _unpack(mfu[0]) * _unpack(mfi[0])
  s = (jnp.sum(mfp * wpa[...], axis=0) + jnp.sum(h2 * wpb[...], axis=0)
       + bpr[0])
  out[...] = s


def _tc_mlp(mf_u, mf_i, mlp_u, mlp_i, W1, b1, W2, b2, Wp, bp):
  grid = (NUM_WORKERS,)
  blk_spec = pl.BlockSpec((1, PAIRS, B_PER_W), lambda g: (g, 0, 0))
  full = lambda shape: pl.BlockSpec(shape, lambda g: tuple(0 for _ in shape))
  return pl.pallas_call(
      _tc_body,
      grid=grid,
      in_specs=[
          blk_spec, blk_spec, blk_spec, blk_spec,
          full((64, 64)),
          full((64, 1)),
          full((32, 64)),
          full((32, 1)),
          full((32, 1)),
          full((32, 1)),
          pl.BlockSpec(memory_space=pltpu.SMEM),
      ],
      out_specs=pl.BlockSpec((B_PER_W,), lambda g: (g,)),
      out_shape=jax.ShapeDtypeStruct((BATCH,), jnp.float32),
  )(mf_u, mf_i, mlp_u, mlp_i, W1, b1.reshape(64, 1), W2, b2.reshape(32, 1),
    Wp[0, :DIM].reshape(DIM, 1), Wp[0, DIM:].reshape(DIM, 1), bp)


def kernel(user, item, mf_user_emb, mf_item_emb, mlp_user_emb, mlp_item_emb,
           W1, b1, W2, b2, Wp, bp):
  user32 = user.astype(jnp.int32)
  item32 = item.astype(jnp.int32)
  sc_detile = _get("sc_detile", _sc_detile_build)
  g0, g1 = sc_detile(mf_user_emb.T, mf_item_emb.T)
  g0, g1 = _tc_tail(mf_user_emb.T, mf_item_emb.T, g0, g1)
  f2 = _tc_detile(mlp_user_emb.T)
  f3 = _tc_detile(mlp_item_emb.T)
  gather = _get("gather", _sc_gather_build)
  mf_u, mf_i, mlp_u, mlp_i = gather(user32, item32, g0, g1, f2, f3)
  r = lambda a: a.reshape(NUM_WORKERS, PAIRS, B_PER_W)
  return _tc_mlp(r(mf_u), r(mf_i), r(mlp_u), r(mlp_i), W1, b1, W2, b2, Wp,
                 bp)


# speedup vs baseline: 1.0308x; 1.0308x over previous
import functools

import jax
import jax.numpy as jnp
from jax import lax
from jax.experimental import pallas as pl
from jax.experimental.pallas import tpu as pltpu
from jax.experimental.pallas import tpu_sc as plsc

BATCH = 16384
DIM = 32
PAIRS = DIM // 2
N_ROWS = 1000000
NUM_WORKERS = 32
B_PER_W = BATCH // NUM_WORKERS
WPW = PAIRS * B_PER_W
LANES = 16

DT_K = 100
DT_L = DT_K * 128
DT_NC = -(-N_ROWS // DT_L)
OUT_ROWS = DT_NC * 2 * DT_K * 8
FLAT_N = OUT_ROWS * 128


def _tc_detile(tabT):
  grid = (DT_NC,)

  def body(t_ref, o_ref):
    x = t_ref[...]
    lo = jax.lax.bitcast_convert_type(
        x[:PAIRS].astype(jnp.bfloat16), jnp.int16).astype(jnp.int32)
    hi = jax.lax.bitcast_convert_type(
        x[PAIRS:].astype(jnp.bfloat16), jnp.int16).astype(jnp.int32)
    w = (lo & 0xFFFF) | (hi << 16)
    y = w.reshape(2, 8, DT_K, 128).transpose(2, 0, 1, 3)
    o_ref[...] = y.reshape(2 * DT_K * 8, 128)

  return pl.pallas_call(
      body,
      grid=grid,
      in_specs=[pl.BlockSpec((DIM, DT_L), lambda c: (0, c))],
      out_specs=pl.BlockSpec((2 * DT_K * 8, 128), lambda c: (c, 0)),
      out_shape=jax.ShapeDtypeStruct((OUT_ROWS, 128), jnp.int32),
  )(tabT).reshape(FLAT_N)


SC_RUN_T = 16
SC_RUN_L = SC_RUN_T * 128
SC_CHUNK_T = 976
SC_RUNS = SC_CHUNK_T // SC_RUN_T
SC_A2 = 7813 * 1024
SC_FLAT = 2 * SC_A2
SC_TAIL_LANE = 7 * SC_CHUNK_T * 128 + SC_RUNS * SC_RUN_L
SC_TAIL_L = N_ROWS - SC_TAIL_LANE


def _sc_detile_build():
  mesh = plsc.VectorSubcoreMesh(core_axis_name="c", subcore_axis_name="s")
  out_sh = jax.ShapeDtypeStruct((SC_FLAT,), jnp.int32)
  fbuf = pltpu.VMEM((8, SC_RUN_L), jnp.float32)
  wbuf = pltpu.VMEM((SC_RUN_T * 1024,), jnp.int32)

  @functools.partial(
      pl.kernel,
      mesh=mesh,
      compiler_params=pltpu.CompilerParams(needs_layout_passes=False),
      out_type=[out_sh, out_sh],
      scratch_types=[
          fbuf, fbuf, fbuf, fbuf,
          wbuf, wbuf,
          pltpu.SemaphoreType.DMA,
          pltpu.SemaphoreType.DMA,
      ],
  )
  def sc_detile(t0, t1, o0, o1, a0, b0, a1, b1, w0, w1, sem_r, sem_w):
    wid = lax.axis_index("s") * 2 + lax.axis_index("c")
    tab_idx = wid // 16
    rest = wid % 16
    a2 = rest // 8
    chunk = rest % 8
    lane_base = chunk * SC_CHUNK_T * 128
    rowA = a2 * 8
    rowB = a2 * 8 + 16

    def for_tab(tab, src, out):
      @pl.when(tab_idx == tab)
      def _():
        out_base = a2 * SC_A2 + (lane_base // 128) * 1024

        def rd(r, ab, bb):
          lane = lane_base + r * SC_RUN_L
          return (
              pltpu.make_async_copy(
                  src.at[pl.ds(rowA, 8), pl.ds(lane, SC_RUN_L)], ab, sem_r),
              pltpu.make_async_copy(
                  src.at[pl.ds(rowB, 8), pl.ds(lane, SC_RUN_L)], bb, sem_r),
          )

        def wr(r, wb):
          return pltpu.make_async_copy(
              wb, out.at[pl.ds(out_base + r * SC_RUN_T * 1024,
                               SC_RUN_T * 1024)], sem_w)

        def vpass(ab, bb, wb, n_lgroups):
          def vbody(j2, _):
            lane = (j2 >> 3) * 128 + (j2 & 7) * 16
            for s in range(8):
              a = ab[s, pl.ds(lane, LANES)]
              b = bb[s, pl.ds(lane, LANES)]
              w = plsc.bitcast(
                  plsc.pack(a, b, format=plsc.PackFormat.INTERLEAVED,
                            preferred_element_type=jnp.bfloat16),
                  jnp.int32)
              wofs = (j2 >> 3) * 1024 + s * 128 + (j2 & 7) * 16
              wb[pl.ds(wofs, LANES)] = w
            return 0

          lax.fori_loop(0, n_lgroups, vbody, 0, unroll=False)

        for d in rd(0, a0, b0):
          d.start()
        for d in rd(1, a1, b1):
          d.start()

        def body(k, _):
          r0 = 2 * k
          r1 = r0 + 1

          @pl.when(k > 0)
          def _():
            wr(r0 - 2, w0).wait()

          for d in rd(r0, a0, b0):
            d.wait()
          vpass(a0, b0, w0, SC_RUN_T * 8)
          wr(r0, w0).start()

          @pl.when(r0 + 2 < SC_RUNS)
          def _():
            for d in rd(r0 + 2, a0, b0):
              d.start()

          @pl.when(k > 0)
          def _():
            wr(r1 - 2, w1).wait()

          for d in rd(r1, a1, b1):
            d.wait()
          vpass(a1, b1, w1, SC_RUN_T * 8)
          wr(r1, w1).start()

          @pl.when(r1 + 2 < SC_RUNS)
          def _():
            for d in rd(r1 + 2, a1, b1):
              d.start()

          return 0

        lax.fori_loop(0, SC_RUNS // 2, body, 0, unroll=False)
        last = SC_RUNS - 1
        wr(last - 2, w0).wait()
        for d in rd(last, a0, b0):
          d.wait()
        vpass(a0, b0, w0, SC_RUN_T * 8)
        wr(last, w0).start()
        wr(last - 1, w1).wait()
        wr(last, w0).wait()

        @pl.when(chunk == 7)
        def _():
          pltpu.sync_copy(
              src.at[pl.ds(rowA, 8), pl.ds(SC_TAIL_LANE, 512)],
              a1.at[:, pl.ds(0, 512)])
          pltpu.sync_copy(
              src.at[pl.ds(rowB, 8), pl.ds(SC_TAIL_LANE, 512)],
              b1.at[:, pl.ds(0, 512)])
          vpass(a1, b1, w1, 512 // LANES)
          tail_q = SC_TAIL_LANE // 128
          pltpu.sync_copy(
              w1.at[pl.ds(0, 4 * 1024)],
              out.at[pl.ds(a2 * SC_A2 + tail_q * 1024, 4 * 1024)])

    for_tab(0, t0, o0)
    for_tab(1, t1, o1)

  return sc_detile


def _tc_tail(tabT0, tabT1, flat0, flat1):
  q_last = 999936 // 128

  def patch_of(x, g):
    lo = jax.lax.bitcast_convert_type(
        x[:PAIRS].astype(jnp.bfloat16), jnp.int16).astype(jnp.int32)
    hi = jax.lax.bitcast_convert_type(
        x[PAIRS:].astype(jnp.bfloat16), jnp.int16).astype(jnp.int32)
    w = (lo & 0xFFFF) | (hi << 16)
    return jnp.where(g == 0, w[:8].reshape(1024), w[8:].reshape(1024))

  def body(t0_ref, t1_ref, _f0, _f1, o0_ref, o1_ref):
    g = pl.program_id(0)
    o0_ref[...] = patch_of(t0_ref[...], g)
    o1_ref[...] = patch_of(t1_ref[...], g)

  out_sh = jax.ShapeDtypeStruct((SC_FLAT,), jnp.int32)
  out_spec = pl.BlockSpec((1024,), lambda g: (g * (SC_A2 // 1024) + q_last,))
  return pl.pallas_call(
      body,
      grid=(2,),
      in_specs=[
          pl.BlockSpec((DIM, 128), lambda g: (0, q_last)),
          pl.BlockSpec((DIM, 128), lambda g: (0, q_last)),
          pl.BlockSpec(memory_space=pl.ANY),
          pl.BlockSpec(memory_space=pl.ANY),
      ],
      out_specs=[out_spec, out_spec],
      out_shape=[out_sh, out_sh],
      input_output_aliases={2: 0, 3: 1},
  )(tabT0, tabT1, flat0, flat1)


def _sc_gather_build():
  mesh = plsc.VectorSubcoreMesh(core_axis_name="c", subcore_axis_name="s")
  out_sh = jax.ShapeDtypeStruct((NUM_WORKERS, 1, WPW), jnp.int32)
  ebuf = pltpu.VMEM((WPW,), jnp.int32)
  dbuf = pltpu.VMEM((WPW,), jnp.int32)

  @functools.partial(
      pl.kernel,
      mesh=mesh,
      out_type=[out_sh, out_sh, out_sh, out_sh],
      scratch_types=[
          pltpu.VMEM((B_PER_W,), jnp.int32),
          pltpu.VMEM((B_PER_W,), jnp.int32),
          ebuf, ebuf, ebuf, ebuf,
          dbuf, dbuf, dbuf, dbuf,
          pltpu.SemaphoreType.DMA,
      ],
  )
  def sc_gather(user_hbm, item_hbm, g0_hbm, g1_hbm, f2_hbm, f3_hbm,
                o0, o1, o2, o3,
                idx_u, idx_i, eu_sc, ei_sc, eu_tc, ei_tc,
                d0, d1, d2, d3, sem):
    wid = lax.axis_index("s") * 2 + lax.axis_index("c")
    base = wid * B_PER_W
    pltpu.sync_copy(user_hbm.at[pl.ds(base, B_PER_W)], idx_u)
    pltpu.sync_copy(item_hbm.at[pl.ds(base, B_PER_W)], idx_i)

    for j in range(B_PER_W // LANES):
      src = pl.ds(j * LANES, LANES)
      iu = idx_u[src]
      ii = idx_i[src]
      bu_sc = ((iu >> 7) << 10) + (iu & 127)
      bi_sc = ((ii >> 7) << 10) + (ii & 127)
      bu_tc = bu_sc + ((iu >> 7) << 10)
      bi_tc = bi_sc + ((ii >> 7) << 10)
      for p in range(PAIRS):
        off_sc = (p // 8) * SC_A2 + (p % 8) * 128
        off_tc = (p // 8) * 1024 + (p % 8) * 128
        dst = pl.ds(p * B_PER_W + j * LANES, LANES)
        eu_sc[dst] = bu_sc + off_sc
        ei_sc[dst] = bi_sc + off_sc
        eu_tc[dst] = bu_tc + off_tc
        ei_tc[dst] = bi_tc + off_tc

    copies = [
        pltpu.async_copy(g0_hbm.at[eu_sc], d0, sem),
        pltpu.async_copy(g1_hbm.at[ei_sc], d1, sem),
        pltpu.async_copy(f2_hbm.at[eu_tc], d2, sem),
        pltpu.async_copy(f3_hbm.at[ei_tc], d3, sem),
    ]
    for c in copies:
      c.wait()
    for buf, out in ((d0, o0), (d1, o1), (d2, o2), (d3, o3)):
      pltpu.sync_copy(buf, out.at[wid, 0])

  return sc_gather


_BUILD_CACHE = {}


def _get(name, builder):
  if name not in _BUILD_CACHE:
    _BUILD_CACHE[name] = builder()
  return _BUILD_CACHE[name]


def _unpack(w):
  lo = jax.lax.bitcast_convert_type(w << 16, jnp.float32)
  hi = jax.lax.bitcast_convert_type(w & jnp.int32(-65536), jnp.float32)
  return jnp.concatenate([lo, hi], axis=0)


def _tc_body(mfu, mfi, mlu, mli, w1, b1c, w2, b2c, wpa, wpb, bpr, out):
  f32 = jnp.float32
  rs = lambda ref: ref[0, 0].reshape(PAIRS, B_PER_W)
  u = _unpack(rs(mlu))
  i = _unpack(rs(mli))
  w1m = w1[...]
  dn = (((1,), (0,)), ((), ()))
  x = (lax.dot_general(w1m[:, :DIM], u, dn, preferred_element_type=f32)
       + lax.dot_general(w1m[:, DIM:], i, dn, preferred_element_type=f32)
       + b1c[...])
  h = jnp.maximum(x, 0.0)
  h2 = jnp.maximum(
      lax.dot_general(w2[...], h, dn, preferred_element_type=f32) + b2c[...],
      0.0)
  mfp = _unpack(rs(mfu)) * _unpack(rs(mfi))
  s = (jnp.sum(mfp * wpa[...], axis=0) + jnp.sum(h2 * wpb[...], axis=0)
       + bpr[0])
  out[...] = s


def _tc_mlp(mf_u, mf_i, mlp_u, mlp_i, W1, b1, W2, b2, Wp, bp):
  grid = (NUM_WORKERS,)
  blk_spec = pl.BlockSpec((1, 1, WPW), lambda g: (g, 0, 0))
  full = lambda shape: pl.BlockSpec(shape, lambda g: tuple(0 for _ in shape))
  return pl.pallas_call(
      _tc_body,
      grid=grid,
      in_specs=[
          blk_spec, blk_spec, blk_spec, blk_spec,
          full((64, 64)),
          full((64, 1)),
          full((32, 64)),
          full((32, 1)),
          full((32, 1)),
          full((32, 1)),
          pl.BlockSpec(memory_space=pltpu.SMEM),
      ],
      out_specs=pl.BlockSpec((B_PER_W,), lambda g: (g,)),
      out_shape=jax.ShapeDtypeStruct((BATCH,), jnp.float32),
  )(mf_u, mf_i, mlp_u, mlp_i, W1, b1.reshape(64, 1), W2, b2.reshape(32, 1),
    Wp[0, :DIM].reshape(DIM, 1), Wp[0, DIM:].reshape(DIM, 1), bp)


def kernel(user, item, mf_user_emb, mf_item_emb, mlp_user_emb, mlp_item_emb,
           W1, b1, W2, b2, Wp, bp):
  user32 = user.astype(jnp.int32)
  item32 = item.astype(jnp.int32)
  sc_detile = _get("sc_detile", _sc_detile_build)
  g0, g1 = sc_detile(mf_user_emb.T, mf_item_emb.T)
  g0, g1 = _tc_tail(mf_user_emb.T, mf_item_emb.T, g0, g1)
  f2 = _tc_detile(mlp_user_emb.T)
  f3 = _tc_detile(mlp_item_emb.T)
  gather = _get("gather", _sc_gather_build)
  mf_u, mf_i, mlp_u, mlp_i = gather(user32, item32, g0, g1, f2, f3)
  return _tc_mlp(mf_u, mf_i, mlp_u, mlp_i, W1, b1, W2, b2, Wp, bp)
